# baseline (device time: 865675 ns/iter reference)
import jax
import jax.numpy as jnp
from jax import lax
from jax.experimental import pallas as pl
from jax.experimental.pallas import tpu as pltpu

CHUNK = 2048
HALF = CHUNK // 2
SUBS = 4
SUB = HALF // SUBS


def kernel(x):
    m, n = x.shape
    n_chunks = m // CHUNK

    def body(x_hbm, out_ref, xbuf, vrecv, hrecv, sbuf,
             cp_sem, vs_sem, vr_sem, hs_sem, hr_sem):
        i = pl.program_id(0)
        my_x = lax.axis_index("x")
        my_y = lax.axis_index("y")
        half_start = my_x * HALF

        def fetch(c):
            return pltpu.make_async_copy(
                x_hbm.at[pl.ds(c * CHUNK, CHUNK), :],
                xbuf.at[lax.rem(c, 2)],
                cp_sem.at[lax.rem(c, 2)],
            )

        def v_copy(c, s):
            return pltpu.make_async_remote_copy(
                src_ref=xbuf.at[
                    lax.rem(c, 2), pl.ds(half_start + s * SUB, SUB), :
                ],
                dst_ref=vrecv.at[lax.rem(c, 3), s],
                send_sem=vs_sem.at[lax.rem(c, 3), s],
                recv_sem=vr_sem.at[lax.rem(c, 3), s],
                device_id=(my_x, 1 - my_y),
                device_id_type=pl.DeviceIdType.MESH,
            )

        def h_copy(c, s):
            return pltpu.make_async_remote_copy(
                src_ref=sbuf.at[lax.rem(c, 2), s],
                dst_ref=hrecv.at[lax.rem(c, 2), s],
                send_sem=hs_sem.at[lax.rem(c, 2), s],
                recv_sem=hr_sem.at[lax.rem(c, 2), s],
                device_id=(1 - my_x, my_y),
                device_id_type=pl.DeviceIdType.MESH,
            )

        @pl.when(i == 0)
        def _prologue():
            barrier_sem = pltpu.get_barrier_semaphore()
            pl.semaphore_signal(
                barrier_sem, inc=1,
                device_id=(my_x, 1 - my_y),
                device_id_type=pl.DeviceIdType.MESH,
            )
            pl.semaphore_signal(
                barrier_sem, inc=1,
                device_id=(1 - my_x, my_y),
                device_id_type=pl.DeviceIdType.MESH,
            )
            pl.semaphore_wait(barrier_sem, 2)
            f = fetch(0)
            f.start()
            f.wait()
            for s in range(SUBS):
                v_copy(0, s).start()

        @pl.when(i + 1 < n_chunks)
        def _prefetch():
            fetch(i + 1).start()

        @pl.when(i > 0)
        def _drain():
            q = lax.rem(i - 1, 2)
            for s in range(SUBS):
                h_copy(i - 1, s).wait()
            out_ref[pl.ds(half_start, HALF), :] = sbuf[q].reshape(HALF, n)
            out_ref[pl.ds((1 - my_x) * HALF, HALF), :] = (
                hrecv[q].reshape(HALF, n)
            )

        @pl.when(i < n_chunks)
        def _pieces():
            p2 = lax.rem(i, 2)
            for s in range(SUBS):
                v_copy(i, s).wait()
                if s == SUBS - 1:

                    @pl.when(i + 1 < n_chunks)
                    def _send_ahead():
                        fetch(i + 1).wait()
                        for t in range(SUBS):
                            v_copy(i + 1, t).start()

                sbuf[p2, s] = (
                    xbuf[p2, pl.ds(half_start + s * SUB, SUB), :]
                    + vrecv[lax.rem(i, 3), s]
                )
                h_copy(i, s).start()

    return pl.pallas_call(
        body,
        grid=(n_chunks + 1,),
        in_specs=[pl.BlockSpec(memory_space=pl.ANY)],
        out_specs=pl.BlockSpec(
            (CHUNK, n),
            lambda i: (jnp.maximum(i - 1, 0), 0),
            memory_space=pltpu.VMEM,
        ),
        out_shape=jax.ShapeDtypeStruct((m, n), x.dtype),
        scratch_shapes=[
            pltpu.VMEM((2, CHUNK, n), x.dtype),
            pltpu.VMEM((3, SUBS, SUB, n), x.dtype),
            pltpu.VMEM((2, SUBS, SUB, n), x.dtype),
            pltpu.VMEM((2, SUBS, SUB, n), x.dtype),
            pltpu.SemaphoreType.DMA((2,)),
            pltpu.SemaphoreType.DMA((3, SUBS)),
            pltpu.SemaphoreType.DMA((3, SUBS)),
            pltpu.SemaphoreType.DMA((2, SUBS)),
            pltpu.SemaphoreType.DMA((2, SUBS)),
        ],
        compiler_params=pltpu.CompilerParams(
            collective_id=0,
            dimension_semantics=("arbitrary",),
            vmem_limit_bytes=96 * 1024 * 1024,
        ),
    )(x)
